# Initial kernel scaffold; baseline (speedup 1.0000x reference)
import jax
import jax.numpy as jnp
from jax.experimental import pallas as pl


def kernel(x, P, log_tau, log_blend):
    raise NotImplementedError("write your pallas kernel here")


# speedup vs baseline: 2.9494x; 2.9494x over previous
import math

import jax
import jax.numpy as jnp
from jax.experimental import pallas as pl
from jax.experimental.pallas import tpu as pltpu

_SQRT_2_OVER_PI = math.sqrt(2.0 / math.pi)


def _phase_a(x_ref, p_ref, sums_ref, counts_ref):
    i = pl.program_id(0)
    x = x_ref[...]
    p0 = p_ref[...]
    pn = p0 / jnp.maximum(
        jnp.sqrt(jnp.sum(p0 * p0, axis=1, keepdims=True)), 1e-12)
    rn = jnp.sqrt(jnp.sum(x * x, axis=1, keepdims=True))
    xn = x / jnp.maximum(rn, 1e-8)
    sims = jnp.clip(
        jax.lax.dot_general(xn, pn, (((1,), (1,)), ((), ())),
                            preferred_element_type=jnp.float32),
        -1.0, 1.0)
    m = jnp.max(sims, axis=1, keepdims=True)
    k_iota = jax.lax.broadcasted_iota(jnp.int32, sims.shape, 1)
    K = sims.shape[1]
    idx = jnp.min(jnp.where(sims >= m, k_iota, K), axis=1)
    e = (k_iota == idx[:, None]).astype(jnp.float32)
    sums_part = jax.lax.dot_general(e, x, (((0,), (0,)), ((), ())),
                                    preferred_element_type=jnp.float32)
    counts_part = jax.lax.dot_general(
        e, jnp.ones((x.shape[0], 1), jnp.float32), (((0,), (0,)), ((), ())),
        preferred_element_type=jnp.float32)

    @pl.when(i == 0)
    def _():
        sums_ref[...] = jnp.zeros_like(sums_ref)
        counts_ref[...] = jnp.zeros_like(counts_ref)

    sums_ref[...] += sums_part
    counts_ref[...] += counts_part


def _phase_b(sums_ref, counts_ref, p_ref, out_ref):
    momentum = 0.999
    p0 = p_ref[...]
    counts = counts_ref[...]
    sums = sums_ref[...]
    centroids = jnp.where(counts > 0.0, sums / jnp.maximum(counts, 1.0), p0)
    new_p = centroids / jnp.maximum(
        jnp.sqrt(jnp.sum(centroids * centroids, axis=1, keepdims=True)), 1e-12)
    p_upd = momentum * p0 + (1.0 - momentum) * new_p
    out_ref[...] = p_upd / jnp.maximum(
        jnp.sqrt(jnp.sum(p_upd * p_upd, axis=1, keepdims=True)), 1e-8)


def _phase_c(lt_ref, lb_ref, x_ref, pn2_ref, out_ref):
    x = x_ref[...]
    pn2 = pn2_ref[...]
    rn = jnp.sqrt(jnp.sum(x * x, axis=1, keepdims=True))
    xn = x / jnp.maximum(rn, 1e-8)
    sims2 = jnp.clip(
        jax.lax.dot_general(xn, pn2, (((1,), (1,)), ((), ())),
                            preferred_element_type=jnp.float32),
        -1.0, 1.0)
    mx = jnp.max(sims2, axis=1, keepdims=True)
    dists = jnp.clip(1.0 - mx, 0.0, 2.0)
    tau = jnp.exp(lt_ref[0])
    alpha = jax.nn.sigmoid(lb_ref[0])
    novelty = 1.0 - jnp.exp(-tau * dists)
    scale = jnp.clip(1.0 - alpha + alpha * novelty, 0.1, 10.0)
    y = x * scale
    out_ref[...] = 0.5 * y * (
        1.0 + jnp.tanh(_SQRT_2_OVER_PI * (y + 0.044715 * y * y * y)))


def kernel(x, P, log_tau, log_blend):
    B, T, D = x.shape
    K = P.shape[0]
    N = B * T
    xf = x.reshape(N, D)
    TT = 512
    n_tiles = N // TT

    sums, counts = pl.pallas_call(
        _phase_a,
        grid=(n_tiles,),
        in_specs=[
            pl.BlockSpec((TT, D), lambda i: (i, 0)),
            pl.BlockSpec((K, D), lambda i: (0, 0)),
        ],
        out_specs=[
            pl.BlockSpec((K, D), lambda i: (0, 0)),
            pl.BlockSpec((K, 1), lambda i: (0, 0)),
        ],
        out_shape=[
            jax.ShapeDtypeStruct((K, D), jnp.float32),
            jax.ShapeDtypeStruct((K, 1), jnp.float32),
        ],
    )(xf, P)

    pn2 = pl.pallas_call(
        _phase_b,
        out_shape=jax.ShapeDtypeStruct((K, D), jnp.float32),
    )(sums, counts, P)

    lt = jnp.reshape(log_tau, (1,))
    lb = jnp.reshape(log_blend, (1,))
    out = pl.pallas_call(
        _phase_c,
        grid=(n_tiles,),
        in_specs=[
            pl.BlockSpec(memory_space=pltpu.SMEM),
            pl.BlockSpec(memory_space=pltpu.SMEM),
            pl.BlockSpec((TT, D), lambda i: (i, 0)),
            pl.BlockSpec((K, D), lambda i: (0, 0)),
        ],
        out_specs=pl.BlockSpec((TT, D), lambda i: (i, 0)),
        out_shape=jax.ShapeDtypeStruct((N, D), jnp.float32),
    )(lt, lb, xf, pn2)

    return out.reshape(B, T, D)
